# baseline (speedup 1.0000x reference)
import jax
import jax.numpy as jnp
from jax import lax
from jax.experimental import pallas as pl
from jax.experimental.pallas import tpu as pltpu
from jax.experimental.pallas import tpu_sc as plsc

N = 100000
D_TOT = 128
D_LAT = 120
HIDDEN = 64

_NW = 32
_CH = 80
_NCH = N // _CH
_ITERS = -(-_NCH // _NW)

_ROWS_BLK = 2000


def _precompute_body(x_ref, w_ref, o1_ref, o2_ref, o3_ref):
    g = jnp.dot(x_ref[...], w_ref[...],
                preferred_element_type=jnp.float32).astype(jnp.bfloat16)
    o1_ref[...] = g[:, 0:HIDDEN]
    o2_ref[...] = g[:, HIDDEN:2 * HIDDEN]
    o3_ref[...] = g[:, 2 * HIDDEN:3 * HIDDEN]


def _update_body(x_ref, s_ref, w0_ref, b1_ref, w2_ref, b2_ref, o_ref):
    xb = x_ref[...]
    h = jnp.tanh(
        jnp.dot(xb, w0_ref[...], preferred_element_type=jnp.float32)
        + s_ref[...].astype(jnp.float32) + b1_ref[...])
    o_ref[...] = xb + jnp.dot(h, w2_ref[...],
                              preferred_element_type=jnp.float32) + b2_ref[...]


def _sc_body(g1, g2, g3, n1, n2, n3, out,
             i1_v, i2_v, i3_v, a_v, b_v, c_v, sem):
    wid = lax.axis_index("s") * 2 + lax.axis_index("c")

    def step(it, carry):
        ch = wid + _NW * it

        @pl.when(ch < _NCH)
        def _():
            base = ch * _CH
            pltpu.sync_copy(n1.at[pl.ds(base, _CH)], i1_v)
            pltpu.sync_copy(n2.at[pl.ds(base, _CH)], i2_v)
            pltpu.sync_copy(n3.at[pl.ds(base, _CH)], i3_v)
            cp1 = pltpu.async_copy(g1.at[i1_v], a_v, sem)
            cp2 = pltpu.async_copy(g2.at[i2_v], b_v, sem)
            cp3 = pltpu.async_copy(g3.at[i3_v], c_v, sem)
            cp1.wait()
            cp2.wait()
            cp3.wait()

            def add_row(i, c2):
                for j in range(HIDDEN // 32):
                    sl = pl.ds(j * 32, 32)
                    a_v[i, sl] = a_v[i, sl] + b_v[i, sl] + c_v[i, sl]
                return c2

            lax.fori_loop(0, _CH, add_row, 0)
            pltpu.sync_copy(a_v, out.at[pl.ds(base, _CH)])

        return carry

    lax.fori_loop(0, _ITERS, step, 0)


_sc_gather_sum = pl.kernel(
    _sc_body,
    out_type=jax.ShapeDtypeStruct((N, HIDDEN), jnp.bfloat16),
    mesh=plsc.VectorSubcoreMesh(core_axis_name="c", subcore_axis_name="s"),
    compiler_params=pltpu.CompilerParams(use_tc_tiling_on_sc=False),
    scratch_types=[
        pltpu.VMEM((_CH,), jnp.int32),
        pltpu.VMEM((_CH,), jnp.int32),
        pltpu.VMEM((_CH,), jnp.int32),
        pltpu.VMEM((_CH, HIDDEN), jnp.bfloat16),
        pltpu.VMEM((_CH, HIDDEN), jnp.bfloat16),
        pltpu.VMEM((_CH, HIDDEN), jnp.bfloat16),
        pltpu.SemaphoreType.DMA,
    ],
)


def kernel(x, neighbour_index, W1, b1, W2, b2):
    W1r = W1.reshape(4, D_TOT, HIDDEN)
    w1cat = jnp.concatenate([W1r[1], W1r[2], W1r[3]], axis=1)
    w0 = W1r[0]
    w2p = jnp.pad(W2, ((0, 0), (0, D_TOT - D_LAT)))
    b2p = jnp.pad(b2, (0, D_TOT - D_LAT)).reshape(1, D_TOT)
    b1r = b1.reshape(1, HIDDEN)
    n1 = neighbour_index[:, 1]
    n2 = neighbour_index[:, 2]
    n3 = neighbour_index[:, 3]

    grid = (N // _ROWS_BLK,)
    g1, g2, g3 = pl.pallas_call(
        _precompute_body,
        grid=grid,
        in_specs=[pl.BlockSpec((_ROWS_BLK, D_TOT), lambda i: (i, 0)),
                  pl.BlockSpec((D_TOT, 3 * HIDDEN), lambda i: (0, 0))],
        out_specs=[pl.BlockSpec((_ROWS_BLK, HIDDEN), lambda i: (i, 0))] * 3,
        out_shape=[jax.ShapeDtypeStruct((N, HIDDEN), jnp.bfloat16)] * 3,
    )(x, w1cat)

    s = _sc_gather_sum(g1, g2, g3, n1, n2, n3)

    out = pl.pallas_call(
        _update_body,
        grid=grid,
        in_specs=[pl.BlockSpec((_ROWS_BLK, D_TOT), lambda i: (i, 0)),
                  pl.BlockSpec((_ROWS_BLK, HIDDEN), lambda i: (i, 0)),
                  pl.BlockSpec((D_TOT, HIDDEN), lambda i: (0, 0)),
                  pl.BlockSpec((1, HIDDEN), lambda i: (0, 0)),
                  pl.BlockSpec((HIDDEN, D_TOT), lambda i: (0, 0)),
                  pl.BlockSpec((1, D_TOT), lambda i: (0, 0))],
        out_specs=pl.BlockSpec((_ROWS_BLK, D_TOT), lambda i: (i, 0)),
        out_shape=jax.ShapeDtypeStruct((N, D_TOT), jnp.float32),
    )(x, s, w0, b1r, w2p, b2p)
    return out


# speedup vs baseline: 1.3317x; 1.3317x over previous
import jax
import jax.numpy as jnp
from jax import lax
from jax.experimental import pallas as pl
from jax.experimental.pallas import tpu as pltpu
from jax.experimental.pallas import tpu_sc as plsc

N = 100000
D_TOT = 128
D_LAT = 120
HIDDEN = 64

_NW = 32
_SC_ROWS = 400
_GS = 80
_NG = _SC_ROWS // _GS
_NSC = N // _SC_ROWS
_ITERS = -(-_NSC // _NW)

_ROWS_BLK = 2000


def _precompute_body(x_ref, w_ref, o1_ref, o2_ref, o3_ref):
    g = jnp.dot(x_ref[...], w_ref[...], preferred_element_type=jnp.float32)
    o1_ref[...] = g[:, 0:HIDDEN]
    o2_ref[...] = g[:, HIDDEN:2 * HIDDEN]
    o3_ref[...] = g[:, 2 * HIDDEN:3 * HIDDEN]


def _update_body(x_ref, s_ref, w0_ref, b1_ref, w2_ref, b2_ref, o_ref):
    xb = x_ref[...]
    h = jnp.tanh(
        jnp.dot(xb, w0_ref[...], preferred_element_type=jnp.float32)
        + s_ref[...] + b1_ref[...])
    o_ref[...] = xb + jnp.dot(h, w2_ref[...],
                              preferred_element_type=jnp.float32) + b2_ref[...]


def _sc_body(g1, g2, g3, nT, out, i_v, a_v, b_v, c_v, sem_i, sem_g, sem_o):
    wid = lax.axis_index("s") * 2 + lax.axis_index("c")

    def issue_idx(it):
        ch = wid + _NW * it

        @pl.when(ch < _NSC)
        def _():
            pltpu.make_async_copy(
                nT.at[:, pl.ds(ch * _SC_ROWS, _SC_ROWS)],
                i_v.at[it % 2], sem_i).start()

    issue_idx(0)

    def body(it, carry):
        ch = wid + _NW * it
        p = it % 2

        @pl.when(ch < _NSC)
        def _():
            base = ch * _SC_ROWS
            pltpu.make_async_copy(
                nT.at[:, pl.ds(0, _SC_ROWS)], i_v.at[p], sem_i).wait()
            @pl.when(it > 0)
            def _():
                pltpu.make_async_copy(
                    a_v, out.at[pl.ds(0, _SC_ROWS)], sem_o).wait()

            cps = []
            for k in range(_NG):
                rs = pl.ds(k * _GS, _GS)
                cps.append(pltpu.async_copy(
                    g1.at[i_v.at[p, 0, rs]], a_v.at[rs], sem_g))
                cps.append(pltpu.async_copy(
                    g2.at[i_v.at[p, 1, rs]], b_v.at[rs], sem_g))
                cps.append(pltpu.async_copy(
                    g3.at[i_v.at[p, 2, rs]], c_v.at[rs], sem_g))
            issue_idx(it + 1)
            for cp in cps:
                cp.wait()

            def add_rows(i, c2):
                r = i * 4
                for dr in range(4):
                    for j in range(HIDDEN // 16):
                        sl = pl.ds(j * 16, 16)
                        plsc.addupdate(a_v.at[r + dr, sl],
                                       b_v[r + dr, sl] + c_v[r + dr, sl])
                return c2

            lax.fori_loop(0, _SC_ROWS // 4, add_rows, 0)
            pltpu.async_copy(a_v, out.at[pl.ds(base, _SC_ROWS)], sem_o)

        return carry

    lax.fori_loop(0, _ITERS, body, 0)
    pltpu.make_async_copy(a_v, out.at[pl.ds(0, _SC_ROWS)], sem_o).wait()


_sc_gather_sum = pl.kernel(
    _sc_body,
    out_type=jax.ShapeDtypeStruct((N, HIDDEN), jnp.float32),
    mesh=plsc.VectorSubcoreMesh(core_axis_name="c", subcore_axis_name="s"),
    compiler_params=pltpu.CompilerParams(use_tc_tiling_on_sc=False),
    scratch_types=[
        pltpu.VMEM((2, 3, _SC_ROWS), jnp.int32),
        pltpu.VMEM((_SC_ROWS, HIDDEN), jnp.float32),
        pltpu.VMEM((_SC_ROWS, HIDDEN), jnp.float32),
        pltpu.VMEM((_SC_ROWS, HIDDEN), jnp.float32),
        pltpu.SemaphoreType.DMA,
        pltpu.SemaphoreType.DMA,
        pltpu.SemaphoreType.DMA,
    ],
)


def kernel(x, neighbour_index, W1, b1, W2, b2):
    W1r = W1.reshape(4, D_TOT, HIDDEN)
    w1cat = jnp.concatenate([W1r[1], W1r[2], W1r[3]], axis=1)
    w0 = W1r[0]
    w2p = jnp.pad(W2, ((0, 0), (0, D_TOT - D_LAT)))
    b2p = jnp.pad(b2, (0, D_TOT - D_LAT)).reshape(1, D_TOT)
    b1r = b1.reshape(1, HIDDEN)
    nT = neighbour_index.T[1:4]

    grid = (N // _ROWS_BLK,)
    g1, g2, g3 = pl.pallas_call(
        _precompute_body,
        grid=grid,
        in_specs=[pl.BlockSpec((_ROWS_BLK, D_TOT), lambda i: (i, 0)),
                  pl.BlockSpec((D_TOT, 3 * HIDDEN), lambda i: (0, 0))],
        out_specs=[pl.BlockSpec((_ROWS_BLK, HIDDEN), lambda i: (i, 0))] * 3,
        out_shape=[jax.ShapeDtypeStruct((N, HIDDEN), jnp.float32)] * 3,
    )(x, w1cat)

    s = _sc_gather_sum(g1, g2, g3, nT)

    out = pl.pallas_call(
        _update_body,
        grid=grid,
        in_specs=[pl.BlockSpec((_ROWS_BLK, D_TOT), lambda i: (i, 0)),
                  pl.BlockSpec((_ROWS_BLK, HIDDEN), lambda i: (i, 0)),
                  pl.BlockSpec((D_TOT, HIDDEN), lambda i: (0, 0)),
                  pl.BlockSpec((1, HIDDEN), lambda i: (0, 0)),
                  pl.BlockSpec((HIDDEN, D_TOT), lambda i: (0, 0)),
                  pl.BlockSpec((1, D_TOT), lambda i: (0, 0))],
        out_specs=pl.BlockSpec((_ROWS_BLK, D_TOT), lambda i: (i, 0)),
        out_shape=jax.ShapeDtypeStruct((N, D_TOT), jnp.float32),
    )(x, s, w0, b1r, w2p, b2p)
    return out
